# baseline (speedup 1.0000x reference)
import functools

import jax
import jax.numpy as jnp
from jax.experimental import pallas as pl

_NUM_EXPERT = 64
_BLOCK = 4096
_NSPLIT = 8
_SLICE = _BLOCK // _NSPLIT


def _router_body(*refs):
    x_refs = refs[:_NSPLIT]
    w_ref = refs[_NSPLIT]
    b_ref = refs[_NSPLIT + 1]
    idx_ref = refs[_NSPLIT + 2]
    val_ref = refs[_NSPLIT + 3]
    w = w_ref[...]
    b = b_ref[...]
    for j in range(_NSPLIT):
        x = x_refs[j][...]
        logits = jax.lax.dot_general(
            x, w, (((1,), (1,)), ((), ())),
            preferred_element_type=jnp.float32) + b

        lane = jax.lax.broadcasted_iota(jnp.int32, logits.shape, 1)
        v1 = jnp.max(logits, axis=1, keepdims=True)
        i1 = jnp.argmax(logits, axis=1).reshape(-1, 1)
        masked = jnp.where(lane == i1, -jnp.inf, logits)
        v2 = jnp.max(masked, axis=1, keepdims=True)
        i2 = jnp.argmax(masked, axis=1).reshape(-1, 1)

        denom = jnp.sum(jnp.exp(logits - v1), axis=1, keepdims=True)
        s1 = 1.0 / denom
        s2 = jnp.exp(v2 - v1) / denom

        rows = pl.ds(j * _SLICE, _SLICE)
        idx_ref[rows, :] = jnp.concatenate([i1, i2], axis=1)
        val_ref[rows, :] = jnp.concatenate([s1, s2], axis=1)


@functools.partial(jax.jit, static_argnames=())
def _run(inp, W, b2d):
    n_tokens, d_model = inp.shape
    grid = (n_tokens // _BLOCK,)
    x_specs = [
        pl.BlockSpec((_SLICE, d_model),
                     functools.partial(lambda i, jj: (i * _NSPLIT + jj, 0), jj=j))
        for j in range(_NSPLIT)
    ]
    return pl.pallas_call(
        _router_body,
        grid=grid,
        in_specs=x_specs + [
            pl.BlockSpec((_NUM_EXPERT, d_model), lambda i: (0, 0)),
            pl.BlockSpec((1, _NUM_EXPERT), lambda i: (0, 0)),
        ],
        out_specs=[
            pl.BlockSpec((_BLOCK, 2), lambda i: (i, 0)),
            pl.BlockSpec((_BLOCK, 2), lambda i: (i, 0)),
        ],
        out_shape=[
            jax.ShapeDtypeStruct((n_tokens, 2), jnp.int32),
            jax.ShapeDtypeStruct((n_tokens, 2), jnp.float32),
        ],
    )(*([inp] * _NSPLIT), W, b2d)


def kernel(inp, W, b):
    idx, val = _run(inp, W, b.reshape(1, -1))
    return idx, val


# speedup vs baseline: 1.7545x; 1.7545x over previous
import functools

import jax
import jax.numpy as jnp
from jax.experimental import pallas as pl

_NUM_EXPERT = 64
_BLOCK = 4096


def _router_body(x_ref, w_ref, b_ref, i1_ref, i2_ref, s1_ref, s2_ref):
    x = x_ref[...]
    w = w_ref[...]
    b = b_ref[...]
    logits = jax.lax.dot_general(
        x, w, (((1,), (1,)), ((), ())),
        preferred_element_type=jnp.float32) + b

    lt = logits.T
    sub = jax.lax.broadcasted_iota(jnp.int32, lt.shape, 0)
    v1 = jnp.max(lt, axis=0, keepdims=True)
    i1 = jnp.argmax(lt, axis=0).reshape(1, -1)
    masked = jnp.where(sub == i1, -jnp.inf, lt)
    v2 = jnp.max(masked, axis=0, keepdims=True)
    i2 = jnp.argmax(masked, axis=0).reshape(1, -1)

    denom = jnp.sum(jnp.exp(lt - v1), axis=0, keepdims=True)
    s1 = 1.0 / denom
    s2 = jnp.exp(v2 - v1) / denom

    i1_ref[...] = i1.reshape(1, 1, _BLOCK)
    i2_ref[...] = i2.reshape(1, 1, _BLOCK)
    s1_ref[...] = s1.reshape(1, 1, _BLOCK)
    s2_ref[...] = s2.reshape(1, 1, _BLOCK)


@jax.jit
def _run(inp, W, b2d):
    n_tokens, d_model = inp.shape
    nblk = n_tokens // _BLOCK
    plane_i = jax.ShapeDtypeStruct((nblk, 1, _BLOCK), jnp.int32)
    plane_f = jax.ShapeDtypeStruct((nblk, 1, _BLOCK), jnp.float32)
    out_spec = pl.BlockSpec((1, 1, _BLOCK), lambda i: (i, 0, 0))
    return pl.pallas_call(
        _router_body,
        grid=(nblk,),
        in_specs=[
            pl.BlockSpec((_BLOCK, d_model), lambda i: (i, 0)),
            pl.BlockSpec((_NUM_EXPERT, d_model), lambda i: (0, 0)),
            pl.BlockSpec((1, _NUM_EXPERT), lambda i: (0, 0)),
        ],
        out_specs=[out_spec, out_spec, out_spec, out_spec],
        out_shape=[plane_i, plane_i, plane_f, plane_f],
    )(inp, W, b2d)


def kernel(inp, W, b):
    i1, i2, s1, s2 = _run(inp, W, b.reshape(1, -1))
    n = inp.shape[0]
    idx = jnp.stack([i1.reshape(n), i2.reshape(n)], axis=-1)
    val = jnp.stack([s1.reshape(n), s2.reshape(n)], axis=-1)
    return idx, val
